# Initial kernel scaffold; baseline (speedup 1.0000x reference)
import functools

import jax
import jax.numpy as jnp
from jax import lax
from jax.experimental import pallas as pl
from jax.experimental.pallas import tpu as pltpu
from jax.experimental.pallas import tpu_sc as plsc

B = 4096
L = 50
R = B * L
WS = 64
FS = 16
NC = 2
NS = 16
NW = NC * NS
RPW = R // NW
CH = 128
NCHUNK = RPW // CH


def _sc_get_embeddings(Wv, pf1, pf2, xi, li, ri, lei, rei):
    mesh = plsc.VectorSubcoreMesh(core_axis_name="c", subcore_axis_name="s")

    @functools.partial(
        pl.kernel,
        mesh=mesh,
        out_type=(
            jax.ShapeDtypeStruct((R, 96), jnp.float32),
            jax.ShapeDtypeStruct((R, 192), jnp.float32),
        ),
        scratch_types=dict(
            xi_v=pltpu.VMEM((NCHUNK, CH), jnp.int32),
            li_v=pltpu.VMEM((NCHUNK, CH), jnp.int32),
            ri_v=pltpu.VMEM((NCHUNK, CH), jnp.int32),
            le_v=pltpu.VMEM((NCHUNK, CH), jnp.int32),
            re_v=pltpu.VMEM((NCHUNK, CH), jnp.int32),
            xrow=pltpu.VMEM((CH, WS), jnp.float32),
            lerow=pltpu.VMEM((CH, WS), jnp.float32),
            rerow=pltpu.VMEM((CH, WS), jnp.float32),
            ldrow=pltpu.VMEM((CH, FS), jnp.float32),
            rdrow=pltpu.VMEM((CH, FS), jnp.float32),
            gsem=pltpu.SemaphoreType.DMA,
        ),
    )
    def k(Wv_h, pf1_h, pf2_h, xi_h, li_h, ri_h, lei_h, rei_h,
          xp_h, xe_h,
          xi_v, li_v, ri_v, le_v, re_v,
          xrow, lerow, rerow, ldrow, rdrow, gsem):
        wid = lax.axis_index("s") * NC + lax.axis_index("c")
        ibase = wid * NCHUNK
        pltpu.sync_copy(xi_h.at[pl.ds(ibase, NCHUNK)], xi_v)
        pltpu.sync_copy(li_h.at[pl.ds(ibase, NCHUNK)], li_v)
        pltpu.sync_copy(ri_h.at[pl.ds(ibase, NCHUNK)], ri_v)
        pltpu.sync_copy(lei_h.at[pl.ds(ibase, NCHUNK)], le_v)
        pltpu.sync_copy(rei_h.at[pl.ds(ibase, NCHUNK)], re_v)

        @pl.loop(0, NCHUNK)
        def _chunk(j):
            r0 = wid * RPW + j * CH
            pltpu.async_copy(Wv_h.at[xi_v.at[j]], xrow, gsem)
            pltpu.async_copy(pf1_h.at[li_v.at[j]], ldrow, gsem)
            pltpu.async_copy(pf2_h.at[ri_v.at[j]], rdrow, gsem)
            pltpu.async_copy(Wv_h.at[le_v.at[j]], lerow, gsem)
            pltpu.async_copy(Wv_h.at[re_v.at[j]], rerow, gsem)
            pltpu.make_async_copy(Wv_h.at[xi_v.at[j]], xrow, gsem).wait()
            pltpu.make_async_copy(pf1_h.at[li_v.at[j]], ldrow, gsem).wait()
            pltpu.make_async_copy(pf2_h.at[ri_v.at[j]], rdrow, gsem).wait()
            pltpu.make_async_copy(Wv_h.at[le_v.at[j]], lerow, gsem).wait()
            pltpu.make_async_copy(Wv_h.at[re_v.at[j]], rerow, gsem).wait()
            pltpu.sync_copy(xrow, xp_h.at[pl.ds(r0, CH), pl.ds(0, WS)])
            pltpu.sync_copy(ldrow, xp_h.at[pl.ds(r0, CH), pl.ds(WS, FS)])
            pltpu.sync_copy(rdrow, xp_h.at[pl.ds(r0, CH), pl.ds(WS + FS, FS)])
            pltpu.sync_copy(xrow, xe_h.at[pl.ds(r0, CH), pl.ds(0, WS)])
            pltpu.sync_copy(lerow, xe_h.at[pl.ds(r0, CH), pl.ds(WS, WS)])
            pltpu.sync_copy(rerow, xe_h.at[pl.ds(r0, CH), pl.ds(2 * WS, WS)])

    return k(Wv, pf1, pf2, xi, li, ri, lei, rei)


def kernel(Wv, pf1, pf2, x, ldist, rdist, leftEnt, rightEnt):
    xi = x.astype(jnp.int32).reshape(R // CH, CH)
    li = ldist.astype(jnp.int32).reshape(R // CH, CH)
    ri = rdist.astype(jnp.int32).reshape(R // CH, CH)
    lei = jnp.repeat(leftEnt.astype(jnp.int32), L).reshape(R // CH, CH)
    rei = jnp.repeat(rightEnt.astype(jnp.int32), L).reshape(R // CH, CH)
    xp, xe = _sc_get_embeddings(Wv, pf1, pf2, xi, li, ri, lei, rei)
    return (xp.reshape(B, L, 96)[:, None], xe.reshape(B, L, 192))


# speedup vs baseline: 2.8686x; 2.8686x over previous
import functools

import jax
import jax.numpy as jnp
from jax import lax
from jax.experimental import pallas as pl
from jax.experimental.pallas import tpu as pltpu
from jax.experimental.pallas import tpu_sc as plsc

B = 4096
L = 50
R = B * L
WS = 64
FS = 16
NC = 2
NS = 16
NW = NC * NS
RPW = R // NW
CH = 128
NCHUNK = RPW // CH


def _sc_get_embeddings(Wv, pf1, pf2, xi, li, ri, lei, rei):
    mesh = plsc.VectorSubcoreMesh(core_axis_name="c", subcore_axis_name="s")

    @functools.partial(
        pl.kernel,
        mesh=mesh,
        out_type=(
            jax.ShapeDtypeStruct((R, 96), jnp.float32),
            jax.ShapeDtypeStruct((R, 192), jnp.float32),
        ),
        scratch_types=dict(
            xi_v=pltpu.VMEM((NCHUNK, CH), jnp.int32),
            li_v=pltpu.VMEM((NCHUNK, CH), jnp.int32),
            ri_v=pltpu.VMEM((NCHUNK, CH), jnp.int32),
            le_v=pltpu.VMEM((NCHUNK, CH), jnp.int32),
            re_v=pltpu.VMEM((NCHUNK, CH), jnp.int32),
            xrow=pltpu.VMEM((CH, WS), jnp.float32),
            lerow=pltpu.VMEM((CH, WS), jnp.float32),
            rerow=pltpu.VMEM((CH, WS), jnp.float32),
            ldrow=pltpu.VMEM((CH, FS), jnp.float32),
            rdrow=pltpu.VMEM((CH, FS), jnp.float32),
            gsem=pltpu.SemaphoreType.DMA,
        ),
        compiler_params=pltpu.CompilerParams(use_tc_tiling_on_sc=False),
    )
    def k(Wv_h, pf1_h, pf2_h, xi_h, li_h, ri_h, lei_h, rei_h,
          xp_h, xe_h,
          xi_v, li_v, ri_v, le_v, re_v,
          xrow, lerow, rerow, ldrow, rdrow, gsem):
        wid = lax.axis_index("s") * NC + lax.axis_index("c")
        pltpu.sync_copy(xi_h.at[wid], xi_v)
        pltpu.sync_copy(li_h.at[wid], li_v)
        pltpu.sync_copy(ri_h.at[wid], ri_v)
        pltpu.sync_copy(lei_h.at[wid], le_v)
        pltpu.sync_copy(rei_h.at[wid], re_v)

        @pl.loop(0, NCHUNK)
        def _chunk(j):
            r0 = wid * RPW + j * CH
            pltpu.async_copy(Wv_h.at[xi_v.at[j]], xrow, gsem)
            pltpu.async_copy(pf1_h.at[li_v.at[j]], ldrow, gsem)
            pltpu.async_copy(pf2_h.at[ri_v.at[j]], rdrow, gsem)
            pltpu.async_copy(Wv_h.at[le_v.at[j]], lerow, gsem)
            pltpu.async_copy(Wv_h.at[re_v.at[j]], rerow, gsem)
            pltpu.make_async_copy(Wv_h.at[xi_v.at[j]], xrow, gsem).wait()
            pltpu.make_async_copy(pf1_h.at[li_v.at[j]], ldrow, gsem).wait()
            pltpu.make_async_copy(pf2_h.at[ri_v.at[j]], rdrow, gsem).wait()
            pltpu.make_async_copy(Wv_h.at[le_v.at[j]], lerow, gsem).wait()
            pltpu.make_async_copy(Wv_h.at[re_v.at[j]], rerow, gsem).wait()
            pltpu.sync_copy(xrow, xp_h.at[pl.ds(r0, CH), pl.ds(0, WS)])
            pltpu.sync_copy(ldrow, xp_h.at[pl.ds(r0, CH), pl.ds(WS, FS)])
            pltpu.sync_copy(rdrow, xp_h.at[pl.ds(r0, CH), pl.ds(WS + FS, FS)])
            pltpu.sync_copy(xrow, xe_h.at[pl.ds(r0, CH), pl.ds(0, WS)])
            pltpu.sync_copy(lerow, xe_h.at[pl.ds(r0, CH), pl.ds(WS, WS)])
            pltpu.sync_copy(rerow, xe_h.at[pl.ds(r0, CH), pl.ds(2 * WS, WS)])

    return k(Wv, pf1, pf2, xi, li, ri, lei, rei)


def kernel(Wv, pf1, pf2, x, ldist, rdist, leftEnt, rightEnt):
    xi = x.astype(jnp.int32).reshape(NW, NCHUNK, CH)
    li = ldist.astype(jnp.int32).reshape(NW, NCHUNK, CH)
    ri = rdist.astype(jnp.int32).reshape(NW, NCHUNK, CH)
    lei = jnp.repeat(leftEnt.astype(jnp.int32), L).reshape(NW, NCHUNK, CH)
    rei = jnp.repeat(rightEnt.astype(jnp.int32), L).reshape(NW, NCHUNK, CH)
    xp, xe = _sc_get_embeddings(Wv, pf1, pf2, xi, li, ri, lei, rei)
    return (xp.reshape(B, L, 96)[:, None], xe.reshape(B, L, 192))


# speedup vs baseline: 3.2776x; 1.1426x over previous
import functools

import jax
import jax.numpy as jnp
from jax import lax
from jax.experimental import pallas as pl
from jax.experimental.pallas import tpu as pltpu
from jax.experimental.pallas import tpu_sc as plsc

B = 4096
L = 50
R = B * L
WS = 64
FS = 16
NC = 2
NS = 16
NW = NC * NS
RPW = R // NW
CH = 128
NCHUNK = RPW // CH


def _sc_get_embeddings(Wv, pf1, pf2, xi, li, ri, lei, rei):
    mesh = plsc.VectorSubcoreMesh(core_axis_name="c", subcore_axis_name="s")

    @functools.partial(
        pl.kernel,
        mesh=mesh,
        out_type=(
            jax.ShapeDtypeStruct((R, 96), jnp.float32),
            jax.ShapeDtypeStruct((R, 192), jnp.float32),
        ),
        scratch_types=dict(
            xi_v=pltpu.VMEM((NCHUNK, CH), jnp.int32),
            li_v=pltpu.VMEM((NCHUNK, CH), jnp.int32),
            ri_v=pltpu.VMEM((NCHUNK, CH), jnp.int32),
            le_v=pltpu.VMEM((NCHUNK, CH), jnp.int32),
            re_v=pltpu.VMEM((NCHUNK, CH), jnp.int32),
            xrow=pltpu.VMEM((3, CH, WS), jnp.float32),
            lerow=pltpu.VMEM((3, CH, WS), jnp.float32),
            rerow=pltpu.VMEM((3, CH, WS), jnp.float32),
            ldrow=pltpu.VMEM((3, CH, FS), jnp.float32),
            rdrow=pltpu.VMEM((3, CH, FS), jnp.float32),
            gsem=pltpu.SemaphoreType.DMA((3,)),
            wsem=pltpu.SemaphoreType.DMA((3,)),
        ),
        compiler_params=pltpu.CompilerParams(use_tc_tiling_on_sc=False),
    )
    def k(Wv_h, pf1_h, pf2_h, xi_h, li_h, ri_h, lei_h, rei_h,
          xp_h, xe_h,
          xi_v, li_v, ri_v, le_v, re_v,
          xrow, lerow, rerow, ldrow, rdrow, gsem, wsem):
        wid = lax.axis_index("s") * NC + lax.axis_index("c")
        pltpu.sync_copy(xi_h.at[wid], xi_v)
        pltpu.sync_copy(li_h.at[wid], li_v)
        pltpu.sync_copy(ri_h.at[wid], ri_v)
        pltpu.sync_copy(lei_h.at[wid], le_v)
        pltpu.sync_copy(rei_h.at[wid], re_v)

        def gather_descs(j, p):
            return (
                pltpu.make_async_copy(Wv_h.at[xi_v.at[j]], xrow.at[p], gsem.at[p]),
                pltpu.make_async_copy(pf1_h.at[li_v.at[j]], ldrow.at[p], gsem.at[p]),
                pltpu.make_async_copy(pf2_h.at[ri_v.at[j]], rdrow.at[p], gsem.at[p]),
                pltpu.make_async_copy(Wv_h.at[le_v.at[j]], lerow.at[p], gsem.at[p]),
                pltpu.make_async_copy(Wv_h.at[re_v.at[j]], rerow.at[p], gsem.at[p]),
            )

        def write_descs(j, p):
            r0 = wid * RPW + j * CH
            return (
                pltpu.make_async_copy(xrow.at[p], xp_h.at[pl.ds(r0, CH), pl.ds(0, WS)], wsem.at[p]),
                pltpu.make_async_copy(ldrow.at[p], xp_h.at[pl.ds(r0, CH), pl.ds(WS, FS)], wsem.at[p]),
                pltpu.make_async_copy(rdrow.at[p], xp_h.at[pl.ds(r0, CH), pl.ds(WS + FS, FS)], wsem.at[p]),
                pltpu.make_async_copy(xrow.at[p], xe_h.at[pl.ds(r0, CH), pl.ds(0, WS)], wsem.at[p]),
                pltpu.make_async_copy(lerow.at[p], xe_h.at[pl.ds(r0, CH), pl.ds(WS, WS)], wsem.at[p]),
                pltpu.make_async_copy(rerow.at[p], xe_h.at[pl.ds(r0, CH), pl.ds(2 * WS, WS)], wsem.at[p]),
            )

        for d in gather_descs(0, 0):
            d.start()

        @pl.loop(0, NCHUNK)
        def _chunk(j):
            p = lax.rem(j, 3)
            pn = lax.rem(j + 1, 3)

            @pl.when(j + 1 < NCHUNK)
            def _prefetch():
                @pl.when(j >= 2)
                def _drain_old_writes():
                    for d in write_descs(j - 2, pn):
                        d.wait()
                for d in gather_descs(j + 1, pn):
                    d.start()

            for d in gather_descs(j, p):
                d.wait()
            for d in write_descs(j, p):
                d.start()

        for d in write_descs(NCHUNK - 2, (NCHUNK - 2) % 3):
            d.wait()
        for d in write_descs(NCHUNK - 1, (NCHUNK - 1) % 3):
            d.wait()

    return k(Wv, pf1, pf2, xi, li, ri, lei, rei)


def kernel(Wv, pf1, pf2, x, ldist, rdist, leftEnt, rightEnt):
    xi = x.astype(jnp.int32).reshape(NW, NCHUNK, CH)
    li = ldist.astype(jnp.int32).reshape(NW, NCHUNK, CH)
    ri = rdist.astype(jnp.int32).reshape(NW, NCHUNK, CH)
    lei = jnp.repeat(leftEnt.astype(jnp.int32), L).reshape(NW, NCHUNK, CH)
    rei = jnp.repeat(rightEnt.astype(jnp.int32), L).reshape(NW, NCHUNK, CH)
    xp, xe = _sc_get_embeddings(Wv, pf1, pf2, xi, li, ri, lei, rei)
    return (xp.reshape(B, L, 96)[:, None], xe.reshape(B, L, 192))


# speedup vs baseline: 3.7956x; 1.1580x over previous
import functools

import jax
import jax.numpy as jnp
from jax import lax
from jax.experimental import pallas as pl
from jax.experimental.pallas import tpu as pltpu
from jax.experimental.pallas import tpu_sc as plsc

B = 4096
L = 50
R = B * L
WS = 64
FS = 16
NC = 2
NS = 16
NW = NC * NS
RPW = R // NW
BPW = B // NW
CH = 128
NCHUNK = RPW // CH
BB = 16


def _sc_gather(Wv, pf1, pf2, xi, li, ri, lei, rei):
    mesh = plsc.VectorSubcoreMesh(core_axis_name="c", subcore_axis_name="s")

    @functools.partial(
        pl.kernel,
        mesh=mesh,
        out_type=(
            jax.ShapeDtypeStruct((R, 128), jnp.float32),
            jax.ShapeDtypeStruct((B, 128), jnp.float32),
        ),
        scratch_types=dict(
            xi_v=pltpu.VMEM((NCHUNK, CH), jnp.int32),
            li_v=pltpu.VMEM((NCHUNK, CH), jnp.int32),
            ri_v=pltpu.VMEM((NCHUNK, CH), jnp.int32),
            le_v=pltpu.VMEM((BPW,), jnp.int32),
            re_v=pltpu.VMEM((BPW,), jnp.int32),
            xrow=pltpu.VMEM((3, CH, WS), jnp.float32),
            ldrow=pltpu.VMEM((3, CH, FS), jnp.float32),
            rdrow=pltpu.VMEM((3, CH, FS), jnp.float32),
            lerow=pltpu.VMEM((BPW, WS), jnp.float32),
            rerow=pltpu.VMEM((BPW, WS), jnp.float32),
            gsem=pltpu.SemaphoreType.DMA((3,)),
            wsem=pltpu.SemaphoreType.DMA((3,)),
            esem=pltpu.SemaphoreType.DMA,
        ),
        compiler_params=pltpu.CompilerParams(use_tc_tiling_on_sc=False),
    )
    def k(Wv_h, pf1_h, pf2_h, xi_h, li_h, ri_h, lei_h, rei_h,
          a_h, e_h,
          xi_v, li_v, ri_v, le_v, re_v,
          xrow, ldrow, rdrow, lerow, rerow, gsem, wsem, esem):
        wid = lax.axis_index("s") * NC + lax.axis_index("c")
        pltpu.sync_copy(xi_h.at[wid], xi_v)
        pltpu.sync_copy(li_h.at[wid], li_v)
        pltpu.sync_copy(ri_h.at[wid], ri_v)
        pltpu.sync_copy(lei_h.at[wid], le_v)
        pltpu.sync_copy(rei_h.at[wid], re_v)

        b0 = wid * BPW
        pltpu.async_copy(Wv_h.at[le_v], lerow, esem)
        pltpu.async_copy(Wv_h.at[re_v], rerow, esem)
        pltpu.make_async_copy(Wv_h.at[le_v], lerow, esem).wait()
        pltpu.make_async_copy(Wv_h.at[re_v], rerow, esem).wait()
        pltpu.async_copy(lerow, e_h.at[pl.ds(b0, BPW), pl.ds(0, WS)], esem)
        pltpu.async_copy(rerow, e_h.at[pl.ds(b0, BPW), pl.ds(WS, WS)], esem)

        def gather_descs(j, p):
            return (
                pltpu.make_async_copy(Wv_h.at[xi_v.at[j]], xrow.at[p], gsem.at[p]),
                pltpu.make_async_copy(pf1_h.at[li_v.at[j]], ldrow.at[p], gsem.at[p]),
                pltpu.make_async_copy(pf2_h.at[ri_v.at[j]], rdrow.at[p], gsem.at[p]),
            )

        def write_descs(j, p):
            r0 = wid * RPW + j * CH
            return (
                pltpu.make_async_copy(xrow.at[p], a_h.at[pl.ds(r0, CH), pl.ds(0, WS)], wsem.at[p]),
                pltpu.make_async_copy(ldrow.at[p], a_h.at[pl.ds(r0, CH), pl.ds(WS, FS)], wsem.at[p]),
                pltpu.make_async_copy(rdrow.at[p], a_h.at[pl.ds(r0, CH), pl.ds(WS + FS, FS)], wsem.at[p]),
            )

        for d in gather_descs(0, 0):
            d.start()

        @pl.loop(0, NCHUNK)
        def _chunk(j):
            p = lax.rem(j, 3)
            pn = lax.rem(j + 1, 3)

            @pl.when(j + 1 < NCHUNK)
            def _prefetch():
                @pl.when(j >= 2)
                def _drain_old_writes():
                    for d in write_descs(j - 2, pn):
                        d.wait()
                for d in gather_descs(j + 1, pn):
                    d.start()

            for d in gather_descs(j, p):
                d.wait()
            for d in write_descs(j, p):
                d.start()

        for d in write_descs(NCHUNK - 2, (NCHUNK - 2) % 3):
            d.wait()
        for d in write_descs(NCHUNK - 1, (NCHUNK - 1) % 3):
            d.wait()
        pltpu.make_async_copy(lerow, e_h.at[pl.ds(b0, BPW), pl.ds(0, WS)], esem).wait()
        pltpu.make_async_copy(rerow, e_h.at[pl.ds(b0, BPW), pl.ds(WS, WS)], esem).wait()

    return k(Wv, pf1, pf2, xi, li, ri, lei, rei)


def _tc_finish(a3, e):
    def body(a_ref, e_ref, xp_ref, xe_ref):
        a = a_ref[...]
        ent = e_ref[...]
        xp_ref[...] = a[:, :, :96]
        e1 = jnp.broadcast_to(ent[:, None, 0:WS], (BB, L, WS))
        e2 = jnp.broadcast_to(ent[:, None, WS:2 * WS], (BB, L, WS))
        xe_ref[...] = jnp.concatenate([a[:, :, 0:WS], e1, e2], axis=-1)

    return pl.pallas_call(
        body,
        out_shape=(
            jax.ShapeDtypeStruct((B, L, 96), jnp.float32),
            jax.ShapeDtypeStruct((B, L, 192), jnp.float32),
        ),
        grid=(B // BB,),
        in_specs=[
            pl.BlockSpec((BB, L, 128), lambda i: (i, 0, 0)),
            pl.BlockSpec((BB, 128), lambda i: (i, 0)),
        ],
        out_specs=(
            pl.BlockSpec((BB, L, 96), lambda i: (i, 0, 0)),
            pl.BlockSpec((BB, L, 192), lambda i: (i, 0, 0)),
        ),
    )(a3, e)


def kernel(Wv, pf1, pf2, x, ldist, rdist, leftEnt, rightEnt):
    xi = x.astype(jnp.int32).reshape(NW, NCHUNK, CH)
    li = ldist.astype(jnp.int32).reshape(NW, NCHUNK, CH)
    ri = rdist.astype(jnp.int32).reshape(NW, NCHUNK, CH)
    lei = leftEnt.astype(jnp.int32).reshape(NW, BPW)
    rei = rightEnt.astype(jnp.int32).reshape(NW, BPW)
    a, e = _sc_gather(Wv, pf1, pf2, xi, li, ri, lei, rei)
    xp, xe = _tc_finish(a.reshape(B, L, 128), e)
    return (xp[:, None], xe)
